# initial kernel scaffold (unmeasured)
import jax
import jax.numpy as jnp
from jax import lax
from jax.experimental import pallas as pl
from jax.experimental.pallas import tpu as pltpu


def kernel(
    x,
):
    def body(*refs):
        pass

    out_shape = jax.ShapeDtypeStruct(..., jnp.float32)
    return pl.pallas_call(body, out_shape=out_shape)(...)


# baseline (device time: 108219 ns/iter reference)
import jax
import jax.numpy as jnp
from jax import lax
from jax.experimental import pallas as pl
from jax.experimental.pallas import tpu as pltpu

M_PER = 4096
N = 1024
HALF = M_PER // 2


def kernel(x):
    def body(x_ref, out_ref, send_a, recv_a, send_b, recv_b):
        my_x = lax.axis_index("x")
        my_y = lax.axis_index("y")
        other_x = 1 - my_x
        other_y = 1 - my_y

        barrier_sem = pltpu.get_barrier_semaphore()
        pl.semaphore_signal(barrier_sem, inc=1, device_id=(other_x, my_y),
                            device_id_type=pl.DeviceIdType.MESH)
        pl.semaphore_signal(barrier_sem, inc=1, device_id=(my_x, other_y),
                            device_id_type=pl.DeviceIdType.MESH)
        pl.semaphore_wait(barrier_sem, 2)

        out_ref[pl.ds(my_x * M_PER, M_PER), :] = x_ref[:, :].astype(jnp.bfloat16)

        my_half_rows = my_x * M_PER + my_y * HALF
        rdma_a = pltpu.make_async_remote_copy(
            src_ref=out_ref.at[pl.ds(my_half_rows, HALF), :],
            dst_ref=out_ref.at[pl.ds(my_half_rows, HALF), :],
            send_sem=send_a,
            recv_sem=recv_a,
            device_id=(other_x, my_y),
            device_id_type=pl.DeviceIdType.MESH,
        )
        rdma_a.start()
        rdma_a.wait()

        recvd_rows = other_x * M_PER + my_y * HALF
        rdma_b = pltpu.make_async_remote_copy(
            src_ref=out_ref.at[pl.ds(recvd_rows, HALF), :],
            dst_ref=out_ref.at[pl.ds(recvd_rows, HALF), :],
            send_sem=send_b,
            recv_sem=recv_b,
            device_id=(my_x, other_y),
            device_id_type=pl.DeviceIdType.MESH,
        )
        rdma_b.start()
        rdma_b.wait()

    return pl.pallas_call(
        body,
        out_shape=jax.ShapeDtypeStruct((2 * M_PER, N), jnp.bfloat16),
        in_specs=[pl.BlockSpec(memory_space=pltpu.VMEM)],
        out_specs=pl.BlockSpec(memory_space=pltpu.VMEM),
        scratch_shapes=[
            pltpu.SemaphoreType.DMA,
            pltpu.SemaphoreType.DMA,
            pltpu.SemaphoreType.DMA,
            pltpu.SemaphoreType.DMA,
        ],
        compiler_params=pltpu.CompilerParams(collective_id=0),
    )(x)


# device time: 66293 ns/iter; 1.6324x vs baseline; 1.6324x over previous
import jax
import jax.numpy as jnp
from jax import lax
from jax.experimental import pallas as pl
from jax.experimental.pallas import tpu as pltpu

M_PER = 4096
N = 1024
HALF = M_PER // 2
K = 16
CH = HALF // K


def kernel(x):
    def body(x_ref, out_ref, send_a, recv_a, send_b, recv_b):
        my_x = lax.axis_index("x")
        my_y = lax.axis_index("y")
        other_x = 1 - my_x
        other_y = 1 - my_y

        barrier_sem = pltpu.get_barrier_semaphore()
        pl.semaphore_signal(barrier_sem, inc=1, device_id=(other_x, my_y),
                            device_id_type=pl.DeviceIdType.MESH)
        pl.semaphore_signal(barrier_sem, inc=1, device_id=(my_x, other_y),
                            device_id_type=pl.DeviceIdType.MESH)
        pl.semaphore_wait(barrier_sem, 2)

        my_half_rows = my_x * M_PER + my_y * HALF
        out_ref[pl.ds(my_half_rows, HALF), :] = (
            x_ref[pl.ds(my_y * HALF, HALF), :].astype(jnp.bfloat16)
        )

        rdma_a = []
        for i in range(K):
            rows = my_half_rows + i * CH
            r = pltpu.make_async_remote_copy(
                src_ref=out_ref.at[pl.ds(rows, CH), :],
                dst_ref=out_ref.at[pl.ds(rows, CH), :],
                send_sem=send_a.at[i],
                recv_sem=recv_a.at[i],
                device_id=(other_x, my_y),
                device_id_type=pl.DeviceIdType.MESH,
            )
            r.start()
            rdma_a.append(r)

        out_ref[pl.ds(my_x * M_PER + other_y * HALF, HALF), :] = (
            x_ref[pl.ds(other_y * HALF, HALF), :].astype(jnp.bfloat16)
        )

        recvd_rows = other_x * M_PER + my_y * HALF
        rdma_b = []
        for i in range(K):
            rdma_a[i].wait_recv()
            rows = recvd_rows + i * CH
            r = pltpu.make_async_remote_copy(
                src_ref=out_ref.at[pl.ds(rows, CH), :],
                dst_ref=out_ref.at[pl.ds(rows, CH), :],
                send_sem=send_b.at[i],
                recv_sem=recv_b.at[i],
                device_id=(my_x, other_y),
                device_id_type=pl.DeviceIdType.MESH,
            )
            r.start()
            rdma_b.append(r)

        for i in range(K):
            rdma_b[i].wait_recv()
        for i in range(K):
            rdma_a[i].wait_send()
            rdma_b[i].wait_send()

    return pl.pallas_call(
        body,
        out_shape=jax.ShapeDtypeStruct((2 * M_PER, N), jnp.bfloat16),
        in_specs=[pl.BlockSpec(memory_space=pltpu.VMEM)],
        out_specs=pl.BlockSpec(memory_space=pltpu.VMEM),
        scratch_shapes=[
            pltpu.SemaphoreType.DMA((K,)),
            pltpu.SemaphoreType.DMA((K,)),
            pltpu.SemaphoreType.DMA((K,)),
            pltpu.SemaphoreType.DMA((K,)),
        ],
        compiler_params=pltpu.CompilerParams(collective_id=0),
    )(x)


# device time: 65015 ns/iter; 1.6645x vs baseline; 1.0197x over previous
import jax
import jax.numpy as jnp
from jax import lax
from jax.experimental import pallas as pl
from jax.experimental.pallas import tpu as pltpu

M_PER = 4096
N = 1024
HALF = M_PER // 2
K = 32
CH = HALF // K


def kernel(x):
    def body(x_ref, out_ref, send_a, recv_a, send_b, recv_b):
        my_x = lax.axis_index("x")
        my_y = lax.axis_index("y")
        other_x = 1 - my_x
        other_y = 1 - my_y

        barrier_sem = pltpu.get_barrier_semaphore()
        pl.semaphore_signal(barrier_sem, inc=1, device_id=(other_x, my_y),
                            device_id_type=pl.DeviceIdType.MESH)
        pl.semaphore_signal(barrier_sem, inc=1, device_id=(my_x, other_y),
                            device_id_type=pl.DeviceIdType.MESH)
        pl.semaphore_wait(barrier_sem, 2)

        my_half_rows = my_x * M_PER + my_y * HALF
        rdma_a = []
        for i in range(K):
            rows = my_half_rows + i * CH
            out_ref[pl.ds(rows, CH), :] = (
                x_ref[pl.ds(my_y * HALF + i * CH, CH), :].astype(jnp.bfloat16)
            )
            r = pltpu.make_async_remote_copy(
                src_ref=out_ref.at[pl.ds(rows, CH), :],
                dst_ref=out_ref.at[pl.ds(rows, CH), :],
                send_sem=send_a.at[i],
                recv_sem=recv_a.at[i],
                device_id=(other_x, my_y),
                device_id_type=pl.DeviceIdType.MESH,
            )
            r.start()
            rdma_a.append(r)

        out_ref[pl.ds(my_x * M_PER + other_y * HALF, HALF), :] = (
            x_ref[pl.ds(other_y * HALF, HALF), :].astype(jnp.bfloat16)
        )

        recvd_rows = other_x * M_PER + my_y * HALF
        rdma_b = []
        for i in range(K):
            rdma_a[i].wait_recv()
            rows = recvd_rows + i * CH
            r = pltpu.make_async_remote_copy(
                src_ref=out_ref.at[pl.ds(rows, CH), :],
                dst_ref=out_ref.at[pl.ds(rows, CH), :],
                send_sem=send_b.at[i],
                recv_sem=recv_b.at[i],
                device_id=(my_x, other_y),
                device_id_type=pl.DeviceIdType.MESH,
            )
            r.start()
            rdma_b.append(r)

        for i in range(K):
            rdma_b[i].wait_recv()
        for i in range(K):
            rdma_a[i].wait_send()
            rdma_b[i].wait_send()

    return pl.pallas_call(
        body,
        out_shape=jax.ShapeDtypeStruct((2 * M_PER, N), jnp.bfloat16),
        in_specs=[pl.BlockSpec(memory_space=pltpu.VMEM)],
        out_specs=pl.BlockSpec(memory_space=pltpu.VMEM),
        scratch_shapes=[
            pltpu.SemaphoreType.DMA((K,)),
            pltpu.SemaphoreType.DMA((K,)),
            pltpu.SemaphoreType.DMA((K,)),
            pltpu.SemaphoreType.DMA((K,)),
        ],
        compiler_params=pltpu.CompilerParams(collective_id=0),
    )(x)
